# baseline (speedup 1.0000x reference)
import functools

import jax
import jax.numpy as jnp
from jax import lax
from jax.experimental import pallas as pl
from jax.experimental.pallas import tpu as pltpu
from jax.experimental.pallas import tpu_sc as plsc

B = 16384
D = 8
NC, NS = 2, 16
NW = NC * NS
BPW = B // NW

_MESH = plsc.VectorSubcoreMesh(core_axis_name="c", subcore_axis_name="s",
                               num_cores=NC, num_subcores=NS)


@functools.partial(
    pl.kernel,
    out_type=(jax.ShapeDtypeStruct((B, D), jnp.float32),
              jax.ShapeDtypeStruct((B, D), jnp.float32)),
    mesh=_MESH,
    scratch_types=(
        pltpu.VMEM((BPW,), jnp.int32),
        pltpu.VMEM((BPW,), jnp.int32),
        pltpu.VMEM((BPW, D), jnp.float32),
        pltpu.VMEM((BPW, D), jnp.float32),
        pltpu.SemaphoreType.DMA,
        pltpu.SemaphoreType.DMA,
    ),
    compiler_params=pltpu.CompilerParams(use_tc_tiling_on_sc=False),
)
def _sc_gather(users, tracks, utab, ttab, ue_out, te_out,
               uidx, tidx, urows, trows, sem_u, sem_t):
    wid = lax.axis_index("s") * NC + lax.axis_index("c")
    base = wid * BPW
    pltpu.sync_copy(users.at[pl.ds(base, BPW)], uidx)
    pltpu.sync_copy(tracks.at[pl.ds(base, BPW)], tidx)
    cu = pltpu.async_copy(utab.at[uidx], urows, sem_u)
    ct = pltpu.async_copy(ttab.at[tidx], trows, sem_t)
    cu.wait()
    ct.wait()
    pltpu.sync_copy(urows, ue_out.at[pl.ds(base, BPW)])
    pltpu.sync_copy(trows, te_out.at[pl.ds(base, BPW)])


BM = 2048


def _mlp_body(ue, te, w1u, w1t, b1, w2, b2, w3, b3, out):
    h = jnp.dot(ue[...], w1u[...], preferred_element_type=jnp.float32)
    h = h + jnp.dot(te[...], w1t[...], preferred_element_type=jnp.float32)
    h = jnp.maximum(h + b1[...], 0.0)
    h = jnp.maximum(jnp.dot(h, w2[...], preferred_element_type=jnp.float32) + b2[...], 0.0)
    o = jnp.dot(h, w3[...], preferred_element_type=jnp.float32) + b3[...]
    out[...] = 1.0 / (1.0 + jnp.exp(-o))


def _tc_mlp(ue, te, W1u, W1t, b1, W2, b2, W3, b3):
    full = lambda shape: pl.BlockSpec(shape, lambda i: (0, 0))
    return pl.pallas_call(
        _mlp_body,
        grid=(B // BM,),
        in_specs=[
            pl.BlockSpec((BM, D), lambda i: (i, 0)),
            pl.BlockSpec((BM, D), lambda i: (i, 0)),
            full((D, 64)), full((D, 64)), full((1, 64)),
            full((64, 32)), full((1, 32)),
            full((32, 1)), full((1, 1)),
        ],
        out_specs=pl.BlockSpec((BM, 1), lambda i: (i, 0)),
        out_shape=jax.ShapeDtypeStruct((B, 1), jnp.float32),
    )(ue, te, W1u, W1t, b1, W2, b2, W3, b3)


def kernel(users, tracks, user_table, track_table, W1, b1, W2, b2, W3, b3):
    ue, te = _sc_gather(users, tracks, user_table, track_table)
    return _tc_mlp(ue, te, W1[:D], W1[D:], b1.reshape(1, 64),
                   W2, b2.reshape(1, 32), W3, b3.reshape(1, 1))
